# baseline (speedup 1.0000x reference)
import functools

import jax
import jax.numpy as jnp
from jax import lax
from jax.experimental import pallas as pl
from jax.experimental.pallas import tpu as pltpu
from jax.experimental.pallas import tpu_sc as plsc

N = 100000
D = 128
E = 128
S = 5

NC = 2
NS = 16
NW = NC * NS

CHUNK = 40
NUM_CHUNKS = N // CHUNK
PER_W = 80
IDXC = S * CHUNK
IDX_PER_W = PER_W * IDXC
NPG = CHUNK // S
LANES = 16
DEPTH = 4


def _sc_gather_sum_body(neigh_p, features, out,
                        idx_all, rows0, rows1, rows2, rows3, acc_a, acc_b,
                        gs0, gs1, gs2, gs3, ws_a, ws_b):
    wid = lax.axis_index("s") * NC + lax.axis_index("c")

    pltpu.sync_copy(neigh_p.at[pl.ds(wid * IDX_PER_W, IDX_PER_W)], idx_all)

    rows = [rows0, rows1, rows2, rows3]
    gsems = [gs0, gs1, gs2, gs3]
    accs = [acc_a, acc_b]
    wsems = [ws_a, ws_b]

    def fire(k, x):
        return [pltpu.async_copy(
                    features.at[idx_all.at[pl.ds(k * IDXC + j * CHUNK, CHUNK)]],
                    rows[x].at[j], gsems[x])
                for j in range(S)]

    def consume(k, x, copies):
        for c in copies:
            c.wait()
        rowsb = rows[x]
        accb = accs[x % 2]

        def sum_row(n, _):
            g = n // NPG
            r0 = S * (n % NPG)
            for l in range(D // LANES):
                sl = pl.ds(l * LANES, LANES)
                v = rowsb[g, r0, sl]
                for j in range(1, S):
                    v = v + rowsb[g, r0 + j, sl]
                accb[n, sl] = v
            return 0
        lax.fori_loop(0, CHUNK, sum_row, 0)
        chunk = wid * PER_W + k
        c_real = jnp.minimum(chunk, NUM_CHUNKS - 1)
        return pltpu.async_copy(accb, out.at[pl.ds(c_real * CHUNK, CHUNK)],
                                wsems[x % 2])

    def quad(q, _):
        k0 = DEPTH * q
        cps = [fire(k0 + x, x) for x in range(DEPTH)]
        wb0 = consume(k0 + 0, 0, cps[0])
        wb1 = consume(k0 + 1, 1, cps[1])
        wb0.wait()
        wb2 = consume(k0 + 2, 2, cps[2])
        wb1.wait()
        wb3 = consume(k0 + 3, 3, cps[3])
        wb2.wait()
        wb3.wait()
        return 0

    lax.fori_loop(0, PER_W // DEPTH, quad, 0)


@jax.jit
def _sc_gather_sum(neigh_p, features):
    mesh = plsc.VectorSubcoreMesh(core_axis_name="c", subcore_axis_name="s")
    return pl.kernel(
        _sc_gather_sum_body,
        out_type=jax.ShapeDtypeStruct((N, D), jnp.float32),
        mesh=mesh,
        scratch_types=[
            pltpu.VMEM((IDX_PER_W,), jnp.int32),
            pltpu.VMEM((S, CHUNK, D), jnp.float32),
            pltpu.VMEM((S, CHUNK, D), jnp.float32),
            pltpu.VMEM((S, CHUNK, D), jnp.float32),
            pltpu.VMEM((S, CHUNK, D), jnp.float32),
            pltpu.VMEM((CHUNK, D), jnp.float32),
            pltpu.VMEM((CHUNK, D), jnp.float32),
            pltpu.SemaphoreType.DMA,
            pltpu.SemaphoreType.DMA,
            pltpu.SemaphoreType.DMA,
            pltpu.SemaphoreType.DMA,
            pltpu.SemaphoreType.DMA,
            pltpu.SemaphoreType.DMA,
        ],
    )(neigh_p, features)


BN = 2048


def _mm_body(f_ref, g_ref, w1_ref, w2_ref, o_ref):
    acc = lax.dot_general(
        w1_ref[...], f_ref[...], (((1,), (1,)), ((), ())),
        preferred_element_type=jnp.float32)
    acc = acc + lax.dot_general(
        w2_ref[...], g_ref[...], (((1,), (1,)), ((), ())),
        preferred_element_type=jnp.float32)
    o_ref[...] = jnp.maximum(acc, 0.0)


@jax.jit
def _tc_matmul(features, nsum, w1, w2s):
    grid = pl.cdiv(N, BN)
    return pl.pallas_call(
        _mm_body,
        grid=(grid,),
        in_specs=[
            pl.BlockSpec((BN, D), lambda i: (i, 0)),
            pl.BlockSpec((BN, D), lambda i: (i, 0)),
            pl.BlockSpec((E, D), lambda i: (0, 0)),
            pl.BlockSpec((E, D), lambda i: (0, 0)),
        ],
        out_specs=pl.BlockSpec((E, BN), lambda i: (0, i)),
        out_shape=jax.ShapeDtypeStruct((E, N), jnp.float32),
    )(features, nsum, w1, w2s)


def kernel(nodes, features, neigh_indices, W):
    del nodes
    neigh_flat = neigh_indices.reshape(-1)
    n_pad_chunks = NW * PER_W - NUM_CHUNKS
    pad = jnp.tile(neigh_flat[-IDXC:], n_pad_chunks)
    neigh_p = jnp.concatenate([neigh_flat, pad])
    w1 = W[:, :D]
    w2s = W[:, D:] * (1.0 / S)
    nsum = _sc_gather_sum(neigh_p, features)
    return _tc_matmul(features, nsum, w1, w2s)


# speedup vs baseline: 1.5290x; 1.5290x over previous
import functools

import jax
import jax.numpy as jnp
from jax import lax
from jax.experimental import pallas as pl
from jax.experimental.pallas import tpu as pltpu
from jax.experimental.pallas import tpu_sc as plsc

N = 100000
D = 128
E = 128
S = 5

NC = 2
NS = 16
NW = NC * NS

CHUNK = 40
NUM_CHUNKS = N // CHUNK
PER_W = 80
IDXC = S * CHUNK
IDX_PER_W = PER_W * IDXC
LANES = 16
DEPTH = 4


def _sc_gather_sum_body(neigh_p, features, out,
                        idx_all, rows0, rows1, rows2, rows3, acc_a, acc_b,
                        gs0, gs1, gs2, gs3, ws_a, ws_b):
    wid = lax.axis_index("s") * NC + lax.axis_index("c")

    pltpu.sync_copy(neigh_p.at[pl.ds(wid * IDX_PER_W, IDX_PER_W)], idx_all)

    rows = [rows0, rows1, rows2, rows3]
    gsems = [gs0, gs1, gs2, gs3]
    accs = [acc_a, acc_b]
    wsems = [ws_a, ws_b]

    def fire(k, x):
        return [pltpu.async_copy(
                    features.at[idx_all.at[pl.ds(k * IDXC + j * CHUNK, CHUNK)]],
                    rows[x].at[j], gsems[x])
                for j in range(S)]

    def consume(k, x, copies):
        for c in copies:
            c.wait()
        rowsb = rows[x]
        accb = accs[x % 2]

        def sum_row(n, _):
            for l in range(D // LANES):
                sl = pl.ds(l * LANES, LANES)
                v = rowsb[0, n, sl]
                for j in range(1, S):
                    v = v + rowsb[j, n, sl]
                accb[n, sl] = v
            return 0
        lax.fori_loop(0, CHUNK, sum_row, 0)
        chunk = wid * PER_W + k
        c_real = jnp.minimum(chunk, NUM_CHUNKS - 1)
        return pltpu.async_copy(accb, out.at[pl.ds(c_real * CHUNK, CHUNK)],
                                wsems[x % 2])

    def quad(q, _):
        k0 = DEPTH * q
        cps = [fire(k0 + x, x) for x in range(DEPTH)]
        wb0 = consume(k0 + 0, 0, cps[0])
        wb1 = consume(k0 + 1, 1, cps[1])
        wb0.wait()
        wb2 = consume(k0 + 2, 2, cps[2])
        wb1.wait()
        wb3 = consume(k0 + 3, 3, cps[3])
        wb2.wait()
        wb3.wait()
        return 0

    lax.fori_loop(0, PER_W // DEPTH, quad, 0)


@jax.jit
def _sc_gather_sum(neigh_p, features):
    mesh = plsc.VectorSubcoreMesh(core_axis_name="c", subcore_axis_name="s")
    return pl.kernel(
        _sc_gather_sum_body,
        out_type=jax.ShapeDtypeStruct((N, D), jnp.float32),
        mesh=mesh,
        scratch_types=[
            pltpu.VMEM((IDX_PER_W,), jnp.int32),
            pltpu.VMEM((S, CHUNK, D), jnp.float32),
            pltpu.VMEM((S, CHUNK, D), jnp.float32),
            pltpu.VMEM((S, CHUNK, D), jnp.float32),
            pltpu.VMEM((S, CHUNK, D), jnp.float32),
            pltpu.VMEM((CHUNK, D), jnp.float32),
            pltpu.VMEM((CHUNK, D), jnp.float32),
            pltpu.SemaphoreType.DMA,
            pltpu.SemaphoreType.DMA,
            pltpu.SemaphoreType.DMA,
            pltpu.SemaphoreType.DMA,
            pltpu.SemaphoreType.DMA,
            pltpu.SemaphoreType.DMA,
        ],
    )(neigh_p, features)


BN = 2048


def _mm_body(f_ref, g_ref, w1_ref, w2_ref, o_ref):
    acc = lax.dot_general(
        w1_ref[...], f_ref[...], (((1,), (1,)), ((), ())),
        preferred_element_type=jnp.float32)
    acc = acc + lax.dot_general(
        w2_ref[...], g_ref[...], (((1,), (1,)), ((), ())),
        preferred_element_type=jnp.float32)
    o_ref[...] = jnp.maximum(acc, 0.0)


@jax.jit
def _tc_matmul(features, nsum, w1, w2s):
    grid = pl.cdiv(N, BN)
    return pl.pallas_call(
        _mm_body,
        grid=(grid,),
        in_specs=[
            pl.BlockSpec((BN, D), lambda i: (i, 0)),
            pl.BlockSpec((BN, D), lambda i: (i, 0)),
            pl.BlockSpec((E, D), lambda i: (0, 0)),
            pl.BlockSpec((E, D), lambda i: (0, 0)),
        ],
        out_specs=pl.BlockSpec((E, BN), lambda i: (0, i)),
        out_shape=jax.ShapeDtypeStruct((E, N), jnp.float32),
    )(features, nsum, w1, w2s)


def kernel(nodes, features, neigh_indices, W):
    del nodes
    neigh_cm = jnp.transpose(
        neigh_indices.reshape(NUM_CHUNKS, CHUNK, S), (0, 2, 1)).reshape(-1)
    n_pad_chunks = NW * PER_W - NUM_CHUNKS
    pad = jnp.tile(neigh_cm[-IDXC:], n_pad_chunks)
    neigh_p = jnp.concatenate([neigh_cm, pad])
    w1 = W[:, :D]
    w2s = W[:, D:] * (1.0 / S)
    nsum = _sc_gather_sum(neigh_p, features)
    return _tc_matmul(features, nsum, w1, w2s)


# speedup vs baseline: 1.6132x; 1.0551x over previous
import functools

import jax
import jax.numpy as jnp
from jax import lax
from jax.experimental import pallas as pl
from jax.experimental.pallas import tpu as pltpu
from jax.experimental.pallas import tpu_sc as plsc

N = 100000
D = 128
E = 128
S = 5

NC = 2
NS = 16
NW = NC * NS

CHUNK = 40
PER_W = 40
IDXC = S * CHUNK
IDX_PER_W = PER_W * IDXC
LANES = 16
DEPTH = 4

BN = 2048
SPLIT_BLOCKS = 25
N1 = SPLIT_BLOCKS * BN
N2 = N - N1
PAD_CHUNKS = NW * PER_W


def _sc_body(n_real_chunks, neigh_p, features, out,
             idx_all, rows0, rows1, rows2, rows3, acc_a, acc_b,
             gs0, gs1, gs2, gs3, ws_a, ws_b):
    wid = lax.axis_index("s") * NC + lax.axis_index("c")

    pltpu.sync_copy(neigh_p.at[pl.ds(wid * IDX_PER_W, IDX_PER_W)], idx_all)

    rows = [rows0, rows1, rows2, rows3]
    gsems = [gs0, gs1, gs2, gs3]
    accs = [acc_a, acc_b]
    wsems = [ws_a, ws_b]

    def fire(k, x):
        return [pltpu.async_copy(
                    features.at[idx_all.at[pl.ds(k * IDXC + j * CHUNK, CHUNK)]],
                    rows[x].at[j], gsems[x])
                for j in range(S)]

    def consume(k, x, copies):
        for c in copies:
            c.wait()
        rowsb = rows[x]
        accb = accs[x % 2]

        def sum_row(n, _):
            for l in range(D // LANES):
                sl = pl.ds(l * LANES, LANES)
                v = rowsb[0, n, sl]
                for j in range(1, S):
                    v = v + rowsb[j, n, sl]
                accb[n, sl] = v
            return 0
        lax.fori_loop(0, CHUNK, sum_row, 0)
        chunk = wid * PER_W + k
        c_real = jnp.minimum(chunk, n_real_chunks - 1)
        return pltpu.async_copy(accb, out.at[pl.ds(c_real * CHUNK, CHUNK)],
                                wsems[x % 2])

    def quad(q, _):
        k0 = DEPTH * q
        cps = [fire(k0 + x, x) for x in range(DEPTH)]
        wb0 = consume(k0 + 0, 0, cps[0])
        wb1 = consume(k0 + 1, 1, cps[1])
        wb0.wait()
        wb2 = consume(k0 + 2, 2, cps[2])
        wb1.wait()
        wb3 = consume(k0 + 3, 3, cps[3])
        wb2.wait()
        wb3.wait()
        return 0

    lax.fori_loop(0, PER_W // DEPTH, quad, 0)


def _make_sc_call(n_nodes):
    n_real_chunks = n_nodes // CHUNK
    mesh = plsc.VectorSubcoreMesh(core_axis_name="c", subcore_axis_name="s")
    return pl.kernel(
        functools.partial(_sc_body, n_real_chunks),
        out_type=jax.ShapeDtypeStruct((n_nodes, D), jnp.float32),
        mesh=mesh,
        scratch_types=[
            pltpu.VMEM((IDX_PER_W,), jnp.int32),
            pltpu.VMEM((S, CHUNK, D), jnp.float32),
            pltpu.VMEM((S, CHUNK, D), jnp.float32),
            pltpu.VMEM((S, CHUNK, D), jnp.float32),
            pltpu.VMEM((S, CHUNK, D), jnp.float32),
            pltpu.VMEM((CHUNK, D), jnp.float32),
            pltpu.VMEM((CHUNK, D), jnp.float32),
            pltpu.SemaphoreType.DMA,
            pltpu.SemaphoreType.DMA,
            pltpu.SemaphoreType.DMA,
            pltpu.SemaphoreType.DMA,
            pltpu.SemaphoreType.DMA,
            pltpu.SemaphoreType.DMA,
        ],
    )


def _mm_body(f_ref, g_ref, w1_ref, w2_ref, o_ref):
    acc = lax.dot_general(
        w1_ref[...], f_ref[...], (((1,), (1,)), ((), ())),
        preferred_element_type=jnp.float32)
    acc = acc + lax.dot_general(
        w2_ref[...], g_ref[...], (((1,), (1,)), ((), ())),
        preferred_element_type=jnp.float32)
    o_ref[...] = jnp.maximum(acc, 0.0)


def _mm_body2(o_in_ref, f_ref, g_ref, w1_ref, w2_ref, o_ref):
    del o_in_ref
    _mm_body(f_ref, g_ref, w1_ref, w2_ref, o_ref)


def _tc_matmul_1(features, nsum1, w1, w2s):
    return pl.pallas_call(
        _mm_body,
        grid=(SPLIT_BLOCKS,),
        in_specs=[
            pl.BlockSpec((BN, D), lambda i: (i, 0)),
            pl.BlockSpec((BN, D), lambda i: (i, 0)),
            pl.BlockSpec((E, D), lambda i: (0, 0)),
            pl.BlockSpec((E, D), lambda i: (0, 0)),
        ],
        out_specs=pl.BlockSpec((E, BN), lambda i: (0, i)),
        out_shape=jax.ShapeDtypeStruct((E, N), jnp.float32),
    )(features, nsum1, w1, w2s)


def _tc_matmul_2(out1, features, nsum2, w1, w2s):
    grid = pl.cdiv(N2, BN)
    return pl.pallas_call(
        _mm_body2,
        grid=(grid,),
        in_specs=[
            pl.BlockSpec(memory_space=pl.ANY),
            pl.BlockSpec((BN, D), lambda i: (i + SPLIT_BLOCKS, 0)),
            pl.BlockSpec((BN, D), lambda i: (i, 0)),
            pl.BlockSpec((E, D), lambda i: (0, 0)),
            pl.BlockSpec((E, D), lambda i: (0, 0)),
        ],
        out_specs=pl.BlockSpec((E, BN), lambda i: (0, i + SPLIT_BLOCKS)),
        out_shape=jax.ShapeDtypeStruct((E, N), jnp.float32),
        input_output_aliases={0: 0},
    )(out1, features, nsum2, w1, w2s)


def kernel(nodes, features, neigh_indices, W):
    del nodes
    num_chunks = N // CHUNK
    neigh_cm = jnp.transpose(
        neigh_indices.reshape(num_chunks, CHUNK, S), (0, 2, 1)).reshape(-1)
    n1c = N1 // CHUNK
    neigh_p1 = neigh_cm[:n1c * IDXC]
    rest = neigh_cm[n1c * IDXC:]
    n2c = N2 // CHUNK
    pad = jnp.tile(neigh_cm[-IDXC:], PAD_CHUNKS - n2c)
    neigh_p2 = jnp.concatenate([rest, pad])

    w1 = W[:, :D]
    w2s = W[:, D:] * (1.0 / S)

    sc1 = _make_sc_call(N1)
    sc2 = _make_sc_call(N2)
    nsum1 = sc1(neigh_p1, features)
    nsum2 = sc2(neigh_p2, features)
    out1 = _tc_matmul_1(features, nsum1, w1, w2s)
    return _tc_matmul_2(out1, features, nsum2, w1, w2s)
